# baseline (speedup 1.0000x reference)
import functools

import jax
import jax.numpy as jnp
from jax import lax
from jax.experimental import pallas as pl
from jax.experimental.pallas import tpu as pltpu
from jax.experimental.pallas import tpu_sc as plsc

N = 10000
E = 160000
HID = 256
HEADS = 4
DH = HID * HEADS

BM = 1000

NC, NS, L = 2, 16, 16
NW = NC * NS
E_PAD = 163840
EPW = E_PAD // NW
NB2 = EPW // 16
NB5 = EPW // 64
NPAD = 10240
SEG = NPAD // NS
NSP = 10112
SEG5 = NSP // NS
NCHUNK = DH // 128

_mesh = plsc.VectorSubcoreMesh(
    core_axis_name="c", subcore_axis_name="s", num_cores=NC, num_subcores=NS
)
_SC_PARAMS = pltpu.CompilerParams(needs_layout_passes=False)


def _wid():
    return lax.axis_index("c") * NS + lax.axis_index("s")


def _proj_body(x_ref, w_ref, b_ref, o_ref):
    o_ref[...] = (
        jnp.dot(x_ref[...], w_ref[...], preferred_element_type=jnp.float32)
        + b_ref[...]
    )


def _proj(x, W, b):
    M, K = x.shape
    Ko = W.shape[1]
    return pl.pallas_call(
        _proj_body,
        grid=(M // BM,),
        in_specs=[
            pl.BlockSpec((BM, K), lambda i: (i, 0)),
            pl.BlockSpec((K, Ko), lambda i: (0, 0)),
            pl.BlockSpec((1, Ko), lambda i: (0, 0)),
        ],
        out_specs=pl.BlockSpec((BM, Ko), lambda i: (i, 0)),
        out_shape=jax.ShapeDtypeStruct((M, Ko), jnp.float32),
    )(x, W, b.reshape(1, Ko))


def _qkvs_body(h_ref, wq, wk, wv, ws, bq, bk, bv, bs, q_o, k_o, v_o, s_o):
    h = h_ref[...]
    for w, b, o in ((wq, bq, q_o), (wk, bk, k_o), (ws, bs, s_o)):
        o[...] = jnp.dot(h, w[...], preferred_element_type=jnp.float32) + b[...]
    v_o[0] = jnp.dot(h, wv[...], preferred_element_type=jnp.float32) + bv[...]


def _sum_qkvs_body(pp_ref, s_ref, wq, wk, wv, ws, bq, bk, bv, bs,
                   q_o, k_o, v_o, s_o):
    h = jnp.maximum(pp_ref[0] + pp_ref[1] + s_ref[...], 0.0)
    for w, b, o in ((wq, bq, q_o), (wk, bk, k_o), (ws, bs, s_o)):
        o[...] = jnp.dot(h, w[...], preferred_element_type=jnp.float32) + b[...]
    v_o[0] = jnp.dot(h, wv[...], preferred_element_type=jnp.float32) + bv[...]


def _qkvs(h_parts, p):
    single = len(h_parts) == 1
    M = N
    K = h_parts[0].shape[-1]
    BN = 128
    bm = BM if single else 400
    grid = (M // bm, DH // BN)
    if single:
        in_specs = [pl.BlockSpec((bm, K), lambda i, j: (i, 0))]
    else:
        in_specs = [
            pl.BlockSpec((NC, bm, K), lambda i, j: (0, i, 0)),
            pl.BlockSpec((bm, K), lambda i, j: (i, 0)),
        ]
    in_specs += [pl.BlockSpec((K, BN), lambda i, j: (0, j)) for _ in range(4)]
    in_specs += [pl.BlockSpec((1, BN), lambda i, j: (0, j)) for _ in range(4)]
    out_specs = [
        pl.BlockSpec((bm, BN), lambda i, j: (i, j)),
        pl.BlockSpec((bm, BN), lambda i, j: (i, j)),
        pl.BlockSpec((1, bm, BN), lambda i, j: (j, i, 0)),
        pl.BlockSpec((bm, BN), lambda i, j: (i, j)),
    ]
    body = _qkvs_body if single else _sum_qkvs_body
    out_shape = [
        jax.ShapeDtypeStruct((M, DH), jnp.float32),
        jax.ShapeDtypeStruct((M, DH), jnp.float32),
        jax.ShapeDtypeStruct((NCHUNK, M, 128), jnp.float32),
        jax.ShapeDtypeStruct((M, DH), jnp.float32),
    ]
    return pl.pallas_call(
        body,
        grid=grid,
        in_specs=in_specs,
        out_specs=out_specs,
        out_shape=out_shape,
    )(*h_parts, p["Wq"], p["Wk"], p["Wv"], p["Ws"],
      p["bq"].reshape(1, DH), p["bk"].reshape(1, DH),
      p["bv"].reshape(1, DH), p["bs"].reshape(1, DH))


def _cls_body(pp_ref, s_ref, w1, b1, w2, b2, o_ref):
    h = pp_ref[0] + pp_ref[1] + s_ref[...]
    r = jnp.maximum(
        jnp.dot(h, w1[...], preferred_element_type=jnp.float32) + b1[...], 0.0)
    o_ref[...] = jnp.dot(r, w2[...], preferred_element_type=jnp.float32) + b2[...]


def _classifier(pp, s, W1, b1, W2, b2):
    M = N
    K = s.shape[1]
    H1 = W1.shape[1]
    O = W2.shape[1]
    return pl.pallas_call(
        _cls_body,
        grid=(M // BM,),
        in_specs=[
            pl.BlockSpec((NC, BM, K), lambda i: (0, i, 0)),
            pl.BlockSpec((BM, K), lambda i: (i, 0)),
            pl.BlockSpec((K, H1), lambda i: (0, 0)),
            pl.BlockSpec((1, H1), lambda i: (0, 0)),
            pl.BlockSpec((H1, O), lambda i: (0, 0)),
            pl.BlockSpec((1, O), lambda i: (0, 0)),
        ],
        out_specs=pl.BlockSpec((BM, O), lambda i: (i, 0)),
        out_shape=jax.ShapeDtypeStruct((M, O), jnp.float32),
    )(pp, s, W1, b1.reshape(1, H1), W2, b2.reshape(1, O))


@functools.partial(
    pl.kernel,
    out_type=jax.ShapeDtypeStruct((HEADS, E_PAD), jnp.float32),
    mesh=_mesh,
    compiler_params=_SC_PARAMS,
    scratch_types=[
        pltpu.VMEM((EPW,), jnp.int32),
        pltpu.VMEM((EPW,), jnp.int32),
        pltpu.VMEM((2, 16, DH), jnp.float32),
        pltpu.VMEM((2, 16, DH), jnp.float32),
        pltpu.VMEM((HEADS, EPW), jnp.float32),
        pltpu.VMEM((HEADS * 16 * 16,), jnp.float32),
        pltpu.SemaphoreType.DMA,
        pltpu.SemaphoreType.DMA,
        pltpu.SemaphoreType.DMA,
        pltpu.SemaphoreType.DMA,
    ],
)
def _k2_alpha(q_hbm, k_hbm, dstg_hbm, srcg_hbm, alpha_hbm,
              dstv, srcv, qbuf, kbuf, abuf, accbuf, qs0, qs1, ks0, ks1):
    e0 = _wid() * EPW
    pltpu.sync_copy(dstg_hbm.at[pl.ds(e0, EPW)], dstv)
    pltpu.sync_copy(srcg_hbm.at[pl.ds(e0, EPW)], srcv)
    iot = lax.iota(jnp.int32, 16)
    qsem = (qs0, qs1)
    ksem = (ks0, ks1)

    def start(b, p):
        pltpu.async_copy(q_hbm.at[dstv.at[pl.ds(b * 16, 16)]],
                         qbuf.at[p], qsem[p])
        pltpu.async_copy(k_hbm.at[srcv.at[pl.ds(b * 16, 16)]],
                         kbuf.at[p], ksem[p])

    start(0, 0)
    start(1, 1)

    def pair_body(g, _):
        for p in range(2):
            b = g * 2 + p
            pltpu.make_async_copy(q_hbm.at[dstv.at[pl.ds(0, 16)]],
                                  qbuf.at[p], qsem[p]).wait()
            pltpu.make_async_copy(k_hbm.at[srcv.at[pl.ds(0, 16)]],
                                  kbuf.at[p], ksem[p]).wait()

            def edge_body(e, _):
                for h in range(HEADS):
                    off = h * HID
                    acc = qbuf[p, e, pl.ds(off, L)] * kbuf[p, e, pl.ds(off, L)]
                    for cc in range(1, HID // L):
                        o2 = off + cc * L
                        acc = acc + (qbuf[p, e, pl.ds(o2, L)]
                                     * kbuf[p, e, pl.ds(o2, L)])
                    accbuf[pl.ds(h * 256 + e * 16, 16)] = acc
                return 0

            lax.fori_loop(0, 16, edge_body, 0)

            @pl.when(g < NB2 // 2 - 1)
            def _():
                start(b + 2, p)

            for h in range(HEADS):
                s = plsc.load_gather(accbuf, [iot * 16 + h * 256])
                for cc in range(1, 16):
                    s = s + plsc.load_gather(accbuf, [iot * 16 + h * 256 + cc])
                abuf[h, pl.ds(b * 16, 16)] = s * (1.0 / 16.0)
        return 0

    lax.fori_loop(0, NB2 // 2, pair_body, 0)
    for h in range(HEADS):
        pltpu.sync_copy(abuf.at[h], alpha_hbm.at[h, pl.ds(e0, EPW)])


@functools.partial(
    pl.kernel,
    out_type=jax.ShapeDtypeStruct((NC, HEADS, NPAD), jnp.float32),
    mesh=_mesh,
    compiler_params=_SC_PARAMS,
    scratch_types=[
        pltpu.VMEM((EPW,), jnp.int32),
        pltpu.VMEM((EPW,), jnp.float32),
        pltpu.VMEM((NPAD,), jnp.float32),
        pltpu.VMEM((NPAD,), jnp.int32),
        pltpu.VMEM((NS, SEG), jnp.float32),
        pltpu.VMEM_SHARED((NS, NPAD), jnp.float32),
    ],
)
def _k3_amax(alpha_hbm, dsts_hbm, amax_part_hbm,
             dstv, av, tbl, claim, comb, shared):
    cid = lax.axis_index("c")
    sid = lax.axis_index("s")
    e0 = _wid() * EPW
    pltpu.sync_copy(dsts_hbm.at[pl.ds(e0, EPW)], dstv)
    lanes = lax.iota(jnp.int32, 16)
    neg = jnp.full((16,), -3.0e38, jnp.float32)

    for h in range(HEADS):
        pltpu.sync_copy(alpha_hbm.at[h, pl.ds(e0, EPW)], av)

        def init_body(i, _):
            tbl[pl.ds(i * 16, 16)] = neg
            return 0

        lax.fori_loop(0, NPAD // 16, init_body, 0)

        def batch_body(b, _):
            d = dstv[pl.ds(b * 16, 16)]
            a = av[pl.ds(b * 16, 16)]

            def cond(rem):
                return plsc.all_reduce_population_count(rem)[0] > 0

            def wbody(rem):
                plsc.store_scatter(claim, [d], lanes, mask=rem)
                got = plsc.load_gather(claim, [d])
                win = jnp.logical_and(rem, got == lanes)
                cur = plsc.load_gather(tbl, [d])
                plsc.store_scatter(tbl, [d], jnp.maximum(cur, a), mask=win)
                return jnp.logical_and(rem, jnp.logical_not(win))

            lax.while_loop(cond, wbody, jnp.full((16,), True))
            return 0

        lax.fori_loop(0, NB2, batch_body, 0)

        pltpu.sync_copy(tbl, shared.at[sid])
        plsc.subcore_barrier()
        for t in range(NS):
            pltpu.sync_copy(shared.at[t, pl.ds(sid * SEG, SEG)], comb.at[t])

        def red_body(i, _):
            m = comb[0, pl.ds(i * 16, 16)]
            for t in range(1, NS):
                m = jnp.maximum(m, comb[t, pl.ds(i * 16, 16)])
            comb[0, pl.ds(i * 16, 16)] = m
            return 0

        lax.fori_loop(0, SEG // 16, red_body, 0)
        pltpu.sync_copy(comb.at[0],
                        amax_part_hbm.at[cid, h, pl.ds(sid * SEG, SEG)])
        plsc.subcore_barrier()


@functools.partial(
    pl.kernel,
    out_type=[
        jax.ShapeDtypeStruct((HEADS, E_PAD), jnp.float32),
        jax.ShapeDtypeStruct((NC, HEADS, NPAD), jnp.float32),
    ],
    mesh=_mesh,
    compiler_params=_SC_PARAMS,
    scratch_types=[
        pltpu.VMEM((EPW,), jnp.int32),
        pltpu.VMEM((EPW,), jnp.float32),
        pltpu.VMEM((NPAD,), jnp.float32),
        pltpu.VMEM((2, NPAD), jnp.float32),
        pltpu.VMEM((NPAD,), jnp.float32),
        pltpu.VMEM((NS, SEG), jnp.float32),
        pltpu.VMEM_SHARED((NS, NPAD), jnp.float32),
    ],
)
def _k4_exdenom(alpha_hbm, dsts_hbm, amax_part_hbm, ex_hbm, denom_part_hbm,
                dstv, av, afold, ftmp, tbl, comb, shared):
    cid = lax.axis_index("c")
    sid = lax.axis_index("s")
    e0 = _wid() * EPW
    pltpu.sync_copy(dsts_hbm.at[pl.ds(e0, EPW)], dstv)
    zero16 = jnp.zeros((16,), jnp.float32)

    for h in range(HEADS):
        pltpu.sync_copy(alpha_hbm.at[h, pl.ds(e0, EPW)], av)
        pltpu.sync_copy(amax_part_hbm.at[0, h], ftmp.at[0])
        pltpu.sync_copy(amax_part_hbm.at[1, h], ftmp.at[1])

        def fold_body(i, _):
            m = jnp.maximum(ftmp[0, pl.ds(i * 16, 16)],
                            ftmp[1, pl.ds(i * 16, 16)])
            m = jnp.where(m < -1.0e38, 0.0, m)
            afold[pl.ds(i * 16, 16)] = m
            tbl[pl.ds(i * 16, 16)] = zero16
            return 0

        lax.fori_loop(0, NPAD // 16, fold_body, 0)

        def batch_body(b, _):
            d = dstv[pl.ds(b * 16, 16)]
            a = av[pl.ds(b * 16, 16)]
            m16 = plsc.load_gather(afold, [d])
            ex = jnp.exp(a - m16)
            av[pl.ds(b * 16, 16)] = ex
            plsc.addupdate_scatter(tbl, [d], ex)
            return 0

        lax.fori_loop(0, NB2, batch_body, 0)
        pltpu.sync_copy(av, ex_hbm.at[h, pl.ds(e0, EPW)])

        pltpu.sync_copy(tbl, shared.at[sid])
        plsc.subcore_barrier()
        for t in range(NS):
            pltpu.sync_copy(shared.at[t, pl.ds(sid * SEG, SEG)], comb.at[t])

        def red_body(i, _):
            m = comb[0, pl.ds(i * 16, 16)]
            for t in range(1, NS):
                m = m + comb[t, pl.ds(i * 16, 16)]
            comb[0, pl.ds(i * 16, 16)] = m
            return 0

        lax.fori_loop(0, SEG // 16, red_body, 0)
        pltpu.sync_copy(comb.at[0],
                        denom_part_hbm.at[cid, h, pl.ds(sid * SEG, SEG)])
        plsc.subcore_barrier()


@functools.partial(
    pl.kernel,
    out_type=jax.ShapeDtypeStruct((HEADS, E_PAD), jnp.float32),
    mesh=_mesh,
    compiler_params=_SC_PARAMS,
    scratch_types=[
        pltpu.VMEM((EPW,), jnp.int32),
        pltpu.VMEM((EPW,), jnp.float32),
        pltpu.VMEM((NPAD,), jnp.float32),
        pltpu.VMEM((NPAD,), jnp.float32),
    ],
)
def _k4b_aw(ex_hbm, dsts_hbm, denom_part_hbm, a_hbm, dstv, exv, dfold, ftmp):
    e0 = _wid() * EPW
    pltpu.sync_copy(dsts_hbm.at[pl.ds(e0, EPW)], dstv)
    for h in range(HEADS):
        pltpu.sync_copy(ex_hbm.at[h, pl.ds(e0, EPW)], exv)
        pltpu.sync_copy(denom_part_hbm.at[0, h], dfold)
        pltpu.sync_copy(denom_part_hbm.at[1, h], ftmp)

        def fold_body(i, _):
            dfold[pl.ds(i * 16, 16)] = (dfold[pl.ds(i * 16, 16)]
                                        + ftmp[pl.ds(i * 16, 16)])
            return 0

        lax.fori_loop(0, NPAD // 16, fold_body, 0)

        def a_body(b, _):
            d = dstv[pl.ds(b * 16, 16)]
            ex = exv[pl.ds(b * 16, 16)]
            dn = plsc.load_gather(dfold, [d])
            exv[pl.ds(b * 16, 16)] = ex / (dn + 1e-16)
            return 0

        lax.fori_loop(0, NB2, a_body, 0)
        pltpu.sync_copy(exv, a_hbm.at[h, pl.ds(e0, EPW)])


@functools.partial(
    pl.kernel,
    out_type=jax.ShapeDtypeStruct((NC, N, DH), jnp.float32),
    mesh=_mesh,
    compiler_params=_SC_PARAMS,
    scratch_types=[
        pltpu.VMEM((NB5, 64), jnp.int32),
        pltpu.VMEM((NB5, 64), jnp.int32),
        pltpu.VMEM((EPW,), jnp.float32),
        pltpu.VMEM((64, 128), jnp.float32),
        pltpu.VMEM((64, 128), jnp.float32),
        pltpu.VMEM((64, 128), jnp.float32),
        pltpu.VMEM_SHARED((NSP, 128), jnp.float32),
        pltpu.SemaphoreType.DMA,
        pltpu.SemaphoreType.DMA,
        pltpu.SemaphoreType.DMA,
    ],
)
def _k5_out(v3_hbm, srcg3_hbm, dsts3_hbm, a_hbm, zeros_hbm, out_hbm,
            sidx, didx, av, vb0, vb1, sb0, shared, gs0, gs1, ss0):
    cid = lax.axis_index("c")
    sid = lax.axis_index("s")
    w = _wid()
    e0 = w * EPW
    pltpu.sync_copy(srcg3_hbm.at[w], sidx)
    pltpu.sync_copy(dsts3_hbm.at[w], didx)
    vbufs = (vb0, vb1)
    gsem = (gs0, gs1)

    for cc in range(NCHUNK):
        h = cc // 2
        if cc % 2 == 0:
            pltpu.sync_copy(a_hbm.at[h, pl.ds(e0, EPW)], av)
        pltpu.sync_copy(zeros_hbm, shared.at[pl.ds(sid * SEG5, SEG5)])
        plsc.subcore_barrier()

        pltpu.async_copy(v3_hbm.at[cc].at[sidx.at[0]], vbufs[0], gsem[0])
        pltpu.async_copy(v3_hbm.at[cc].at[sidx.at[1]], vbufs[1], gsem[1])

        def pair_body(g, _):
            for p in range(2):
                b = g * 2 + p
                pltpu.make_async_copy(v3_hbm.at[cc].at[sidx.at[0]],
                                      vbufs[p], gsem[p]).wait()

                @pl.when(b >= 1)
                def _():
                    pltpu.make_async_copy(sb0, shared.at[didx.at[0]],
                                          ss0).wait()

                def e_body(e, _):
                    e2 = e * 2
                    sp0 = plsc.load_gather(
                        av, [jnp.zeros((16,), jnp.int32) + b * 64 + e2])
                    sp1 = plsc.load_gather(
                        av, [jnp.zeros((16,), jnp.int32) + b * 64 + e2 + 1])
                    for j in range(8):
                        sb0[e2, pl.ds(j * 16, 16)] = (
                            vbufs[p][e2, pl.ds(j * 16, 16)] * sp0)
                    for j in range(8):
                        sb0[e2 + 1, pl.ds(j * 16, 16)] = (
                            vbufs[p][e2 + 1, pl.ds(j * 16, 16)] * sp1)
                    return 0

                lax.fori_loop(0, 32, e_body, 0)
                pltpu.async_copy(sb0, shared.at[didx.at[b]], ss0, add=True)

                @pl.when(g < NB5 // 2 - 1)
                def _():
                    pltpu.async_copy(v3_hbm.at[cc].at[sidx.at[b + 2]],
                                     vbufs[p], gsem[p])
            return 0

        lax.fori_loop(0, NB5 // 2, pair_body, 0)
        pltpu.make_async_copy(sb0, shared.at[didx.at[0]], ss0).wait()
        plsc.subcore_barrier()

        @pl.when(sid < NS - 1)
        def _():
            pltpu.sync_copy(
                shared.at[pl.ds(sid * SEG5, SEG5)],
                out_hbm.at[cid, pl.ds(sid * SEG5, SEG5), pl.ds(cc * 128, 128)])

        @pl.when(sid == NS - 1)
        def _():
            pltpu.sync_copy(
                shared.at[pl.ds(sid * SEG5, N - (NS - 1) * SEG5)],
                out_hbm.at[cid, pl.ds(sid * SEG5, N - (NS - 1) * SEG5),
                           pl.ds(cc * 128, 128)])

        plsc.subcore_barrier()


def kernel(x, edge_index, params):
    src = edge_index[0]
    dst = edge_index[1]
    pad = E_PAD - E
    srcg = jnp.concatenate([src, jnp.zeros((pad,), jnp.int32)])
    dstg = jnp.concatenate([dst, jnp.zeros((pad,), jnp.int32)])
    dsts = jnp.concatenate(
        [dst, N + (jnp.arange(pad, dtype=jnp.int32) % (NSP - N))])
    srcg3 = srcg.reshape(NW, NB5, 64)
    dsts3 = dsts.reshape(NW, NB5, 64)
    zrows = jnp.zeros((SEG5, 128), jnp.float32)

    h = _proj(x, params["proj_W"], params["proj_b"])
    parts = [h]
    for i in range(3):
        p = params["layers"][i]
        q, k, v, s = _qkvs(parts, p)
        alpha = _k2_alpha(q, k, dstg, srcg)
        amax_part = _k3_amax(alpha, dsts)
        ex, denom_part = _k4_exdenom(alpha, dsts, amax_part)
        aw = _k4b_aw(ex, dsts, denom_part)
        out_parts = _k5_out(v, srcg3, dsts3, aw, zrows)
        parts = [out_parts, s]
    return _classifier(parts[0], parts[1],
                       params["cls_W1"], params["cls_b1"],
                       params["cls_W2"], params["cls_b2"])


# speedup vs baseline: 1.8471x; 1.8471x over previous
import functools

import jax
import jax.numpy as jnp
from jax import lax
from jax.experimental import pallas as pl
from jax.experimental.pallas import tpu as pltpu
from jax.experimental.pallas import tpu_sc as plsc

N = 10000
E = 160000
HID = 256
HEADS = 4
DH = HID * HEADS

BM = 1000

NC, NS, L = 2, 16, 16
NW = NC * NS
E_PAD = 163840
EPW = E_PAD // NW
NB2 = EPW // 16
NB5 = EPW // 64
NPAD = 10240
SEG = NPAD // NS
NSP = NPAD
SEG5 = SEG
NCHUNK = DH // 128

_mesh = plsc.VectorSubcoreMesh(
    core_axis_name="c", subcore_axis_name="s", num_cores=NC, num_subcores=NS
)
_SC_PARAMS = pltpu.CompilerParams(needs_layout_passes=False)


def _wid():
    return lax.axis_index("c") * NS + lax.axis_index("s")


def _proj_body(x_ref, w_ref, b_ref, o_ref):
    o_ref[...] = (
        jnp.dot(x_ref[...], w_ref[...], preferred_element_type=jnp.float32)
        + b_ref[...]
    )


def _proj(x, W, b):
    M, K = x.shape
    Ko = W.shape[1]
    return pl.pallas_call(
        _proj_body,
        grid=(M // BM,),
        in_specs=[
            pl.BlockSpec((BM, K), lambda i: (i, 0)),
            pl.BlockSpec((K, Ko), lambda i: (0, 0)),
            pl.BlockSpec((1, Ko), lambda i: (0, 0)),
        ],
        out_specs=pl.BlockSpec((BM, Ko), lambda i: (i, 0)),
        out_shape=jax.ShapeDtypeStruct((M, Ko), jnp.float32),
    )(x, W, b.reshape(1, Ko))


def _qkvs_body(h_ref, wq, wk, wv, ws, bq, bk, bv, bs, q_o, k_o, v_o, s_o):
    h = h_ref[...]
    for w, b, o in ((wq, bq, q_o), (wk, bk, k_o), (ws, bs, s_o)):
        o[...] = jnp.dot(h, w[...], preferred_element_type=jnp.float32) + b[...]
    v_o[0] = jnp.dot(h, wv[...], preferred_element_type=jnp.float32) + bv[...]


def _sum_qkvs_body(pp_ref, s_ref, wq, wk, wv, ws, bq, bk, bv, bs,
                   q_o, k_o, v_o, s_o):
    h = jnp.maximum(pp_ref[0] + pp_ref[1] + s_ref[...], 0.0)
    for w, b, o in ((wq, bq, q_o), (wk, bk, k_o), (ws, bs, s_o)):
        o[...] = jnp.dot(h, w[...], preferred_element_type=jnp.float32) + b[...]
    v_o[0] = jnp.dot(h, wv[...], preferred_element_type=jnp.float32) + bv[...]


def _qkvs(h_parts, p):
    single = len(h_parts) == 1
    M = N
    K = h_parts[0].shape[-1]
    BN = 128
    bm = BM if single else 400
    grid = (M // bm, DH // BN)
    if single:
        in_specs = [pl.BlockSpec((bm, K), lambda i, j: (i, 0))]
    else:
        in_specs = [
            pl.BlockSpec((NC, bm, K), lambda i, j: (0, i, 0)),
            pl.BlockSpec((bm, K), lambda i, j: (i, 0)),
        ]
    in_specs += [pl.BlockSpec((K, BN), lambda i, j: (0, j)) for _ in range(4)]
    in_specs += [pl.BlockSpec((1, BN), lambda i, j: (0, j)) for _ in range(4)]
    out_specs = [
        pl.BlockSpec((bm, BN), lambda i, j: (i, j)),
        pl.BlockSpec((bm, BN), lambda i, j: (i, j)),
        pl.BlockSpec((1, bm, BN), lambda i, j: (j, i, 0)),
        pl.BlockSpec((bm, BN), lambda i, j: (i, j)),
    ]
    body = _qkvs_body if single else _sum_qkvs_body
    out_shape = [
        jax.ShapeDtypeStruct((M, DH), jnp.float32),
        jax.ShapeDtypeStruct((M, DH), jnp.float32),
        jax.ShapeDtypeStruct((NCHUNK, M, 128), jnp.float32),
        jax.ShapeDtypeStruct((M, DH), jnp.float32),
    ]
    return pl.pallas_call(
        body,
        grid=grid,
        in_specs=in_specs,
        out_specs=out_specs,
        out_shape=out_shape,
    )(*h_parts, p["Wq"], p["Wk"], p["Wv"], p["Ws"],
      p["bq"].reshape(1, DH), p["bk"].reshape(1, DH),
      p["bv"].reshape(1, DH), p["bs"].reshape(1, DH))


def _cls_body(pp_ref, s_ref, w1, b1, w2, b2, o_ref):
    h = pp_ref[0] + pp_ref[1] + s_ref[...]
    r = jnp.maximum(
        jnp.dot(h, w1[...], preferred_element_type=jnp.float32) + b1[...], 0.0)
    o_ref[...] = jnp.dot(r, w2[...], preferred_element_type=jnp.float32) + b2[...]


def _classifier(pp, s, W1, b1, W2, b2):
    M = N
    K = s.shape[1]
    H1 = W1.shape[1]
    O = W2.shape[1]
    return pl.pallas_call(
        _cls_body,
        grid=(M // BM,),
        in_specs=[
            pl.BlockSpec((NC, BM, K), lambda i: (0, i, 0)),
            pl.BlockSpec((BM, K), lambda i: (i, 0)),
            pl.BlockSpec((K, H1), lambda i: (0, 0)),
            pl.BlockSpec((1, H1), lambda i: (0, 0)),
            pl.BlockSpec((H1, O), lambda i: (0, 0)),
            pl.BlockSpec((1, O), lambda i: (0, 0)),
        ],
        out_specs=pl.BlockSpec((BM, O), lambda i: (i, 0)),
        out_shape=jax.ShapeDtypeStruct((M, O), jnp.float32),
    )(pp, s, W1, b1.reshape(1, H1), W2, b2.reshape(1, O))


@functools.partial(
    pl.kernel,
    out_type=jax.ShapeDtypeStruct((HEADS, E_PAD), jnp.float32),
    mesh=_mesh,
    compiler_params=_SC_PARAMS,
    scratch_types=[
        pltpu.VMEM((EPW,), jnp.int32),
        pltpu.VMEM((EPW,), jnp.int32),
        pltpu.VMEM((2, 16, DH), jnp.float32),
        pltpu.VMEM((2, 16, DH), jnp.float32),
        pltpu.VMEM((HEADS, EPW), jnp.float32),
        pltpu.VMEM((HEADS * 16 * 16,), jnp.float32),
        pltpu.SemaphoreType.DMA,
        pltpu.SemaphoreType.DMA,
        pltpu.SemaphoreType.DMA,
        pltpu.SemaphoreType.DMA,
    ],
)
def _k2_alpha(q_hbm, k_hbm, dstg_hbm, srcg_hbm, alpha_hbm,
              dstv, srcv, qbuf, kbuf, abuf, accbuf, qs0, qs1, ks0, ks1):
    e0 = _wid() * EPW
    pltpu.sync_copy(dstg_hbm.at[pl.ds(e0, EPW)], dstv)
    pltpu.sync_copy(srcg_hbm.at[pl.ds(e0, EPW)], srcv)
    iot = lax.iota(jnp.int32, 16)
    qsem = (qs0, qs1)
    ksem = (ks0, ks1)

    def start(b, p):
        pltpu.async_copy(q_hbm.at[dstv.at[pl.ds(b * 16, 16)]],
                         qbuf.at[p], qsem[p])
        pltpu.async_copy(k_hbm.at[srcv.at[pl.ds(b * 16, 16)]],
                         kbuf.at[p], ksem[p])

    start(0, 0)
    start(1, 1)

    def pair_body(g, _):
        for p in range(2):
            b = g * 2 + p
            pltpu.make_async_copy(q_hbm.at[dstv.at[pl.ds(0, 16)]],
                                  qbuf.at[p], qsem[p]).wait()
            pltpu.make_async_copy(k_hbm.at[srcv.at[pl.ds(0, 16)]],
                                  kbuf.at[p], ksem[p]).wait()

            def edge_body(e, _):
                for h in range(HEADS):
                    off = h * HID
                    acc = qbuf[p, e, pl.ds(off, L)] * kbuf[p, e, pl.ds(off, L)]
                    for cc in range(1, HID // L):
                        o2 = off + cc * L
                        acc = acc + (qbuf[p, e, pl.ds(o2, L)]
                                     * kbuf[p, e, pl.ds(o2, L)])
                    accbuf[pl.ds(h * 256 + e * 16, 16)] = acc
                return 0

            lax.fori_loop(0, 16, edge_body, 0)

            @pl.when(g < NB2 // 2 - 1)
            def _():
                start(b + 2, p)

            for h in range(HEADS):
                s = plsc.load_gather(accbuf, [iot * 16 + h * 256])
                for cc in range(1, 16):
                    s = s + plsc.load_gather(accbuf, [iot * 16 + h * 256 + cc])
                abuf[h, pl.ds(b * 16, 16)] = s * (1.0 / 16.0)
        return 0

    lax.fori_loop(0, NB2 // 2, pair_body, 0)
    for h in range(HEADS):
        pltpu.sync_copy(abuf.at[h], alpha_hbm.at[h, pl.ds(e0, EPW)])


@functools.partial(
    pl.kernel,
    out_type=jax.ShapeDtypeStruct((NC, HEADS, NPAD), jnp.float32),
    mesh=_mesh,
    compiler_params=_SC_PARAMS,
    scratch_types=[
        pltpu.VMEM((EPW,), jnp.int32),
        pltpu.VMEM((EPW,), jnp.float32),
        pltpu.VMEM((NPAD,), jnp.float32),
        pltpu.VMEM((NPAD,), jnp.int32),
        pltpu.VMEM((NS, SEG), jnp.float32),
        pltpu.VMEM_SHARED((NS, NPAD), jnp.float32),
    ],
)
def _k3_amax(alpha_hbm, dsts_hbm, amax_part_hbm,
             dstv, av, tbl, claim, comb, shared):
    cid = lax.axis_index("c")
    sid = lax.axis_index("s")
    e0 = _wid() * EPW
    pltpu.sync_copy(dsts_hbm.at[pl.ds(e0, EPW)], dstv)
    lanes = lax.iota(jnp.int32, 16)
    neg = jnp.full((16,), -3.0e38, jnp.float32)

    for h in range(HEADS):
        pltpu.sync_copy(alpha_hbm.at[h, pl.ds(e0, EPW)], av)

        def init_body(i, _):
            tbl[pl.ds(i * 16, 16)] = neg
            return 0

        lax.fori_loop(0, NPAD // 16, init_body, 0)

        def batch_body(b, _):
            d = dstv[pl.ds(b * 16, 16)]
            a = av[pl.ds(b * 16, 16)]

            def cond(rem):
                return plsc.all_reduce_population_count(rem)[0] > 0

            def wbody(rem):
                plsc.store_scatter(claim, [d], lanes, mask=rem)
                got = plsc.load_gather(claim, [d])
                win = jnp.logical_and(rem, got == lanes)
                cur = plsc.load_gather(tbl, [d])
                plsc.store_scatter(tbl, [d], jnp.maximum(cur, a), mask=win)
                return jnp.logical_and(rem, jnp.logical_not(win))

            lax.while_loop(cond, wbody, jnp.full((16,), True))
            return 0

        lax.fori_loop(0, NB2, batch_body, 0)

        pltpu.sync_copy(tbl, shared.at[sid])
        plsc.subcore_barrier()
        for t in range(NS):
            pltpu.sync_copy(shared.at[t, pl.ds(sid * SEG, SEG)], comb.at[t])

        def red_body(i, _):
            m = comb[0, pl.ds(i * 16, 16)]
            for t in range(1, NS):
                m = jnp.maximum(m, comb[t, pl.ds(i * 16, 16)])
            comb[0, pl.ds(i * 16, 16)] = m
            return 0

        lax.fori_loop(0, SEG // 16, red_body, 0)
        pltpu.sync_copy(comb.at[0],
                        amax_part_hbm.at[cid, h, pl.ds(sid * SEG, SEG)])
        plsc.subcore_barrier()


@functools.partial(
    pl.kernel,
    out_type=[
        jax.ShapeDtypeStruct((HEADS, E_PAD), jnp.float32),
        jax.ShapeDtypeStruct((NC, HEADS, NPAD), jnp.float32),
    ],
    mesh=_mesh,
    compiler_params=_SC_PARAMS,
    scratch_types=[
        pltpu.VMEM((EPW,), jnp.int32),
        pltpu.VMEM((EPW,), jnp.float32),
        pltpu.VMEM((NPAD,), jnp.float32),
        pltpu.VMEM((2, NPAD), jnp.float32),
        pltpu.VMEM((NPAD,), jnp.float32),
        pltpu.VMEM((NS, SEG), jnp.float32),
        pltpu.VMEM_SHARED((NS, NPAD), jnp.float32),
    ],
)
def _k4_exdenom(alpha_hbm, dsts_hbm, amax_part_hbm, ex_hbm, denom_part_hbm,
                dstv, av, afold, ftmp, tbl, comb, shared):
    cid = lax.axis_index("c")
    sid = lax.axis_index("s")
    e0 = _wid() * EPW
    pltpu.sync_copy(dsts_hbm.at[pl.ds(e0, EPW)], dstv)
    zero16 = jnp.zeros((16,), jnp.float32)

    for h in range(HEADS):
        pltpu.sync_copy(alpha_hbm.at[h, pl.ds(e0, EPW)], av)
        pltpu.sync_copy(amax_part_hbm.at[0, h], ftmp.at[0])
        pltpu.sync_copy(amax_part_hbm.at[1, h], ftmp.at[1])

        def fold_body(i, _):
            m = jnp.maximum(ftmp[0, pl.ds(i * 16, 16)],
                            ftmp[1, pl.ds(i * 16, 16)])
            m = jnp.where(m < -1.0e38, 0.0, m)
            afold[pl.ds(i * 16, 16)] = m
            tbl[pl.ds(i * 16, 16)] = zero16
            return 0

        lax.fori_loop(0, NPAD // 16, fold_body, 0)

        def batch_body(b, _):
            d = dstv[pl.ds(b * 16, 16)]
            a = av[pl.ds(b * 16, 16)]
            m16 = plsc.load_gather(afold, [d])
            ex = jnp.exp(a - m16)
            av[pl.ds(b * 16, 16)] = ex
            plsc.addupdate_scatter(tbl, [d], ex)
            return 0

        lax.fori_loop(0, NB2, batch_body, 0)
        pltpu.sync_copy(av, ex_hbm.at[h, pl.ds(e0, EPW)])

        pltpu.sync_copy(tbl, shared.at[sid])
        plsc.subcore_barrier()
        for t in range(NS):
            pltpu.sync_copy(shared.at[t, pl.ds(sid * SEG, SEG)], comb.at[t])

        def red_body(i, _):
            m = comb[0, pl.ds(i * 16, 16)]
            for t in range(1, NS):
                m = m + comb[t, pl.ds(i * 16, 16)]
            comb[0, pl.ds(i * 16, 16)] = m
            return 0

        lax.fori_loop(0, SEG // 16, red_body, 0)
        pltpu.sync_copy(comb.at[0],
                        denom_part_hbm.at[cid, h, pl.ds(sid * SEG, SEG)])
        plsc.subcore_barrier()


@functools.partial(
    pl.kernel,
    out_type=jax.ShapeDtypeStruct((HEADS, E_PAD), jnp.float32),
    mesh=_mesh,
    compiler_params=_SC_PARAMS,
    scratch_types=[
        pltpu.VMEM((EPW,), jnp.int32),
        pltpu.VMEM((EPW,), jnp.float32),
        pltpu.VMEM((NPAD,), jnp.float32),
        pltpu.VMEM((NPAD,), jnp.float32),
    ],
)
def _k4b_aw(ex_hbm, dsts_hbm, denom_part_hbm, a_hbm, dstv, exv, dfold, ftmp):
    e0 = _wid() * EPW
    pltpu.sync_copy(dsts_hbm.at[pl.ds(e0, EPW)], dstv)
    for h in range(HEADS):
        pltpu.sync_copy(ex_hbm.at[h, pl.ds(e0, EPW)], exv)
        pltpu.sync_copy(denom_part_hbm.at[0, h], dfold)
        pltpu.sync_copy(denom_part_hbm.at[1, h], ftmp)

        def fold_body(i, _):
            dfold[pl.ds(i * 16, 16)] = (dfold[pl.ds(i * 16, 16)]
                                        + ftmp[pl.ds(i * 16, 16)])
            return 0

        lax.fori_loop(0, NPAD // 16, fold_body, 0)

        def a_body(b, _):
            d = dstv[pl.ds(b * 16, 16)]
            ex = exv[pl.ds(b * 16, 16)]
            dn = plsc.load_gather(dfold, [d])
            exv[pl.ds(b * 16, 16)] = ex / (dn + 1e-16)
            return 0

        lax.fori_loop(0, NB2, a_body, 0)
        pltpu.sync_copy(exv, a_hbm.at[h, pl.ds(e0, EPW)])


@functools.partial(
    pl.kernel,
    out_type=jax.ShapeDtypeStruct((NC, N, DH), jnp.float32),
    mesh=_mesh,
    compiler_params=_SC_PARAMS,
    scratch_types=[
        pltpu.VMEM((NB5, 64), jnp.int32),
        pltpu.VMEM((NB5, 64), jnp.int32),
        pltpu.VMEM((EPW,), jnp.float32),
        pltpu.VMEM((64, 128), jnp.float32),
        pltpu.VMEM((64, 128), jnp.float32),
        pltpu.VMEM_SHARED((NSP, 128), jnp.float32),
        pltpu.SemaphoreType.DMA,
        pltpu.SemaphoreType.DMA,
    ],
)
def _k5_out(v3_hbm, srcg3_hbm, dsts3_hbm, a_hbm, zeros_hbm, out_hbm,
            sidx, didx, av, vb0, vb1, shared, gs0, gs1):
    cid = lax.axis_index("c")
    sid = lax.axis_index("s")
    w = _wid()
    e0 = w * EPW
    pltpu.sync_copy(srcg3_hbm.at[w], sidx)
    pltpu.sync_copy(dsts3_hbm.at[w], didx)
    vbufs = (vb0, vb1)
    gsem = (gs0, gs1)

    for cc in range(NCHUNK):
        h = cc // 2
        if cc % 2 == 0:
            pltpu.sync_copy(a_hbm.at[h, pl.ds(e0, EPW)], av)
        pltpu.sync_copy(zeros_hbm, shared.at[pl.ds(sid * SEG5, SEG5)])
        plsc.subcore_barrier()

        pltpu.async_copy(v3_hbm.at[cc].at[sidx.at[0]], vbufs[0], gsem[0])

        def pair_body(g, _):
            for p in range(2):
                b = g * 2 + p
                pltpu.make_async_copy(v3_hbm.at[cc].at[sidx.at[0]],
                                      vbufs[p], gsem[p]).wait()

                @pl.when(b + 1 < NB5)
                def _():
                    pltpu.async_copy(v3_hbm.at[cc].at[sidx.at[b + 1]],
                                     vbufs[1 - p], gsem[1 - p])

                def e_body(e, _):
                    sp = plsc.load_gather(
                        av, [jnp.zeros((16,), jnp.int32) + b * 64 + e])
                    for j in range(8):
                        vbufs[p][e, pl.ds(j * 16, 16)] = (
                            vbufs[p][e, pl.ds(j * 16, 16)] * sp)
                    return 0

                lax.fori_loop(0, 64, e_body, 0)
                pltpu.sync_copy(vbufs[p], shared.at[didx.at[b]], add=True)
            return 0

        lax.fori_loop(0, NB5 // 2, pair_body, 0)
        plsc.subcore_barrier()

        @pl.when(sid < NS - 1)
        def _():
            pltpu.sync_copy(
                shared.at[pl.ds(sid * SEG5, SEG5)],
                out_hbm.at[cid, pl.ds(sid * SEG5, SEG5), pl.ds(cc * 128, 128)])

        @pl.when(sid == NS - 1)
        def _():
            pltpu.sync_copy(
                shared.at[pl.ds(sid * SEG5, N - (NS - 1) * SEG5)],
                out_hbm.at[cid, pl.ds(sid * SEG5, N - (NS - 1) * SEG5),
                           pl.ds(cc * 128, 128)])

        plsc.subcore_barrier()


def kernel(x, edge_index, params):
    src = edge_index[0]
    dst = edge_index[1]
    pad = E_PAD - E
    spread = jnp.arange(pad, dtype=jnp.int32) * 37 % N
    srcg = jnp.concatenate([src, spread])
    dstg = jnp.concatenate([dst, spread])
    dsts = jnp.concatenate(
        [dst, N + (jnp.arange(pad, dtype=jnp.int32) % (NPAD - N))])
    srcg3 = srcg.reshape(NW, NB5, 64)
    dsts3 = dsts.reshape(NW, NB5, 64)
    zrows = jnp.zeros((SEG5, 128), jnp.float32)

    h = _proj(x, params["proj_W"], params["proj_b"])
    parts = [h]
    for i in range(3):
        p = params["layers"][i]
        q, k, v, s = _qkvs(parts, p)
        alpha = _k2_alpha(q, k, dstg, srcg)
        amax_part = _k3_amax(alpha, dsts)
        ex, denom_part = _k4_exdenom(alpha, dsts, amax_part)
        aw = _k4b_aw(ex, dsts, denom_part)
        out_parts = _k5_out(v, srcg3, dsts3, aw, zrows)
        parts = [out_parts, s]
    return _classifier(parts[0], parts[1],
                       params["cls_W1"], params["cls_b1"],
                       params["cls_W2"], params["cls_b2"])
